# Initial kernel scaffold; baseline (speedup 1.0000x reference)
import functools

import jax
import jax.numpy as jnp
from jax import lax
from jax.experimental import pallas as pl
from jax.experimental.pallas import tpu as pltpu
from jax.experimental.pallas import tpu_sc as plsc

K = 32
N = 4096
NK = N * K


_KNN_R = 256


def _knn_body(rows_ref, pc_ref, pcT_ref, nbr_ref, xyz_ref, ef_ref):
    b = pl.program_id(0)
    rows = rows_ref[0]
    pc_all = pc_ref[0]
    pcT = pcT_ref[0]

    r0 = rows[:, 0:1]
    r1 = rows[:, 1:2]
    r2 = rows[:, 2:3]
    sq_r = (r0 * r0 + r1 * r1) + r2 * r2
    c0 = pcT[0:1, :]
    c1 = pcT[1:2, :]
    c2 = pcT[2:3, :]
    sq_c = (c0 * c0 + c1 * c1) + c2 * c2

    mm = lax.dot_general(rows, pcT, (((1,), (0,)), ((), ())),
                         preferred_element_type=jnp.float32)
    d = (sq_r + sq_c) - 2.0 * mm

    iota = lax.broadcasted_iota(jnp.int32, d.shape, 1)
    big = jnp.int32(N)
    for t in range(K):
        m = jnp.min(d, axis=1, keepdims=True)
        ismin = d == m
        j = jnp.min(jnp.where(ismin, iota, big), axis=1, keepdims=True)
        sel = ismin & (iota == j)
        onehot = sel.astype(jnp.float32)
        xyz_t = lax.dot_general(onehot, pc_all, (((1,), (0,)), ((), ())),
                                preferred_element_type=jnp.float32,
                                precision=lax.Precision.HIGHEST)
        nbr_ref[0, :, t:t + 1] = j + b * N
        xyz_ref[0, t, :, :] = xyz_t
        ef_ref[0, t, :, :] = xyz_t - rows
        d = jnp.where(sel, jnp.float32(jnp.inf), d)


def _knn(pc, pcT):
    B = pc.shape[0]
    R = _KNN_R
    grid = (B, N // R)
    return pl.pallas_call(
        _knn_body,
        grid=grid,
        in_specs=[
            pl.BlockSpec((1, R, 3), lambda b, i: (b, i, 0)),
            pl.BlockSpec((1, N, 3), lambda b, i: (b, 0, 0)),
            pl.BlockSpec((1, 3, N), lambda b, i: (b, 0, 0)),
        ],
        out_specs=[
            pl.BlockSpec((1, R, K), lambda b, i: (b, i, 0)),
            pl.BlockSpec((1, K, R, 3), lambda b, i: (b, 0, i, 0)),
            pl.BlockSpec((1, K, R, 3), lambda b, i: (b, 0, i, 0)),
        ],
        out_shape=[
            jax.ShapeDtypeStruct((B, N, K), jnp.int32),
            jax.ShapeDtypeStruct((B, K, N, 3), jnp.float32),
            jax.ShapeDtypeStruct((B, K, N, 3), jnp.float32),
        ],
    )(pc, pc, pcT)


_SC_CHUNK = 128


def _sc_gather(table, idx):
    M = idx.shape[0]
    D = table.shape[1]
    info = plsc.get_sparse_core_info()
    NW = info.num_cores * info.num_subcores
    m_per_w = M // NW
    CH = _SC_CHUNK
    n_iter = m_per_w // CH
    mesh = plsc.VectorSubcoreMesh(core_axis_name="c", subcore_axis_name="s")

    @functools.partial(
        pl.kernel,
        out_type=jax.ShapeDtypeStruct((M, D), jnp.float32),
        mesh=mesh,
        scratch_types=[
            pltpu.VMEM((CH,), jnp.int32),
            pltpu.VMEM((CH, D), jnp.float32),
            pltpu.SemaphoreType.DMA,
        ],
    )
    def gather_kernel(table_hbm, idx_hbm, out_hbm, idx_v, rows_v, sem):
        wid = lax.axis_index("s") * info.num_cores + lax.axis_index("c")
        base = wid * m_per_w

        def body(i, carry):
            off = base + i * CH
            pltpu.sync_copy(idx_hbm.at[pl.ds(off, CH)], idx_v)
            pltpu.async_copy(table_hbm.at[idx_v], rows_v, sem).wait()
            pltpu.sync_copy(rows_v, out_hbm.at[pl.ds(off, CH)])
            return carry

        lax.fori_loop(0, n_iter, body, 0)

    return gather_kernel(table, idx)


_CONV_R = 2048
_EPS = 1e-5


def _lrelu(x):
    return jnp.where(x >= 0, x, 0.1 * x)


def _norm(x, stats_blk, g, be):
    m_rows = jnp.float32(NK)
    s1 = stats_blk[0:1, :]
    s2 = stats_blk[1:2, :]
    mean = s1 / m_rows
    var = s2 / m_rows - mean * mean
    inv = lax.rsqrt(var + _EPS)
    return (x - mean) * inv * g + be


def _chain(G, E, ws, stats, upto):
    X = (lax.dot_general(G, ws["W1g"], (((1,), (0,)), ((), ())),
                         preferred_element_type=jnp.float32)
         + lax.dot_general(E, ws["W1e"], (((1,), (0,)), ((), ())),
                           preferred_element_type=jnp.float32)
         + ws["b1"])
    for j in (2, 3):
        if upto < j:
            break
        Xn = _lrelu(_norm(X, stats[j - 2], ws["g%d" % (j - 1)], ws["be%d" % (j - 1)]))
        X = lax.dot_general(Xn, ws["W%d" % j], (((1,), (0,)), ((), ())),
                            preferred_element_type=jnp.float32) + ws["b%d" % j]
    return X


def _w_names(stage):
    names = ["W1g", "W1e", "b1"]
    for j in (2, 3):
        if stage >= j:
            names += ["g%d" % (j - 1), "be%d" % (j - 1), "W%d" % j, "b%d" % j]
    return names


def _stats_pass_body(stage, refs):
    i = pl.program_id(1)
    G_ref, E_ref = refs[0], refs[1]
    pos = 2
    n_stats = stage - 1
    stats = [refs[pos + k][0] for k in range(n_stats)]
    pos += n_stats
    w_refs = refs[pos:-1]
    out_ref = refs[-1]
    names = _w_names(stage)
    ws = {n: w_refs[k][...] for k, n in enumerate(names)}
    X = _chain(G_ref[0], E_ref[0], ws, stats, stage)
    s1 = jnp.sum(X, axis=0, keepdims=True)
    s2 = jnp.sum(X * X, axis=0, keepdims=True)
    new = jnp.concatenate([s1, s2], axis=0)

    @pl.when(i == 0)
    def _():
        out_ref[0] = new

    @pl.when(i > 0)
    def _():
        out_ref[0] = out_ref[0] + new


def _pool_pass_body(refs):
    G_ref, E_ref = refs[0], refs[1]
    stats = [refs[2][0], refs[3][0], refs[4][0]]
    w_refs = refs[5:-1]
    out_ref = refs[-1]
    names = _w_names(3) + ["g3", "be3"]
    ws = {n: w_refs[k][...] for k, n in enumerate(names)}
    X = _chain(G_ref[0], E_ref[0], ws, stats[:2], 3)
    Xn = _lrelu(_norm(X, stats[2], ws["g3"], ws["be3"]))
    c = Xn.shape[-1]
    P = jnp.max(Xn.reshape(_CONV_R // K, K, c), axis=1)
    out_ref[0] = P


def _conv_layer(G, E, p):
    B, _, C = G.shape
    cout = p["W1"].shape[1]
    R = _CONV_R
    grid = (B, NK // R)

    W1g = p["W1"][:C, :]
    W1e = p["W1"][C:, :]

    def wmap(stage, with_g3):
        arrs = [W1g, W1e, p["b1"].reshape(1, cout)]
        for j in (2, 3):
            if stage >= j:
                arrs += [p["g%d" % (j - 1)].reshape(1, cout),
                         p["be%d" % (j - 1)].reshape(1, cout),
                         p["W%d" % j], p["b%d" % j].reshape(1, cout)]
        if with_g3:
            arrs += [p["g3"].reshape(1, cout), p["be3"].reshape(1, cout)]
        return arrs

    def full_spec(a):
        nd = a.ndim
        return pl.BlockSpec(a.shape, lambda b, i, _nd=nd: (0,) * _nd)

    ge_specs = [
        pl.BlockSpec((1, R, C), lambda b, i: (b, i, 0)),
        pl.BlockSpec((1, R, 3), lambda b, i: (b, i, 0)),
    ]
    stat_spec = pl.BlockSpec((1, 2, cout), lambda b, i: (b, 0, 0))

    stats = []
    for stage in (1, 2, 3):
        warrs = wmap(stage, False)
        body = functools.partial(_stats_pass_body, stage)

        def wrapped(*refs, _body=body):
            _body(refs)

        st = pl.pallas_call(
            wrapped,
            grid=grid,
            in_specs=(ge_specs + [stat_spec] * (stage - 1)
                      + [full_spec(a) for a in warrs]),
            out_specs=stat_spec,
            out_shape=jax.ShapeDtypeStruct((B, 2, cout), jnp.float32),
        )(G, E, *stats, *warrs)
        stats.append(st)

    warrs = wmap(3, True)

    def wrapped_pool(*refs):
        _pool_pass_body(refs)

    pooled = pl.pallas_call(
        wrapped_pool,
        grid=grid,
        in_specs=ge_specs + [stat_spec] * 3 + [full_spec(a) for a in warrs],
        out_specs=pl.BlockSpec((1, R // K, cout), lambda b, i: (b, i, 0)),
        out_shape=jax.ShapeDtypeStruct((B, N, cout), jnp.float32),
    )(G, E, *stats, *warrs)
    return pooled


def kernel(pc, params):
    B = pc.shape[0]
    pcT = jnp.swapaxes(pc, 1, 2)
    nbr, xyz_t, ef_t = _knn(pc, pcT)
    edges = nbr.reshape(-1)
    G1 = jnp.transpose(xyz_t, (0, 2, 1, 3)).reshape(B, NK, 3)
    E = jnp.transpose(ef_t, (0, 2, 1, 3)).reshape(B, NK, 3)
    edge_feats = E.reshape(B * NK, 3)

    pooled1 = _conv_layer(G1, E, params["conv1"])
    G2 = _sc_gather(pooled1.reshape(B * N, -1), edges).reshape(B, NK, -1)
    pooled2 = _conv_layer(G2, E, params["conv2"])
    G3 = _sc_gather(pooled2.reshape(B * N, -1), edges).reshape(B, NK, -1)
    pooled3 = _conv_layer(G3, E, params["conv3"])

    x = jnp.swapaxes(pooled3, 1, 2)
    return (x, edges, edge_feats)


# speedup vs baseline: 1.6417x; 1.6417x over previous
import functools

import jax
import jax.numpy as jnp
from jax import lax
from jax.experimental import pallas as pl
from jax.experimental.pallas import tpu as pltpu
from jax.experimental.pallas import tpu_sc as plsc

K = 32
N = 4096
NK = N * K


_KNN_R = 128


def _knn_body(rows_ref, pc_ref, pcT_ref, nbr_ref, xyz_ref, ef_ref):
    b = pl.program_id(0)
    rows = rows_ref[0]
    pc_all = pc_ref[0]
    pcT = pcT_ref[0]

    r0 = rows[:, 0:1]
    r1 = rows[:, 1:2]
    r2 = rows[:, 2:3]
    sq_r = (r0 * r0 + r1 * r1) + r2 * r2
    c0 = pcT[0:1, :]
    c1 = pcT[1:2, :]
    c2 = pcT[2:3, :]
    sq_c = (c0 * c0 + c1 * c1) + c2 * c2

    mm = lax.dot_general(rows, pcT, (((1,), (0,)), ((), ())),
                         preferred_element_type=jnp.float32)
    d = (sq_r + sq_c) - 2.0 * mm

    iota = lax.broadcasted_iota(jnp.int32, d.shape, 1)
    iota_k = lax.broadcasted_iota(jnp.int32, (d.shape[0], K), 1)
    big = jnp.int32(N)

    def it(t, carry):
        d, nbr_acc = carry
        m = jnp.min(d, axis=1, keepdims=True)
        ismin = d == m
        j = jnp.min(jnp.where(ismin, iota, big), axis=1, keepdims=True)
        sel = ismin & (iota == j)
        onehot = sel.astype(jnp.float32)
        xyz_t = lax.dot_general(onehot, pc_all, (((1,), (0,)), ((), ())),
                                preferred_element_type=jnp.float32,
                                precision=lax.Precision.HIGHEST)
        xyz_ref[0, pl.ds(t, 1), :, :] = xyz_t[None]
        ef_ref[0, pl.ds(t, 1), :, :] = (xyz_t - rows)[None]
        nbr_acc = jnp.where(iota_k == t, j, nbr_acc)
        d = jnp.where(sel, jnp.float32(jnp.inf), d)
        return d, nbr_acc

    nbr0 = jnp.zeros((d.shape[0], K), jnp.int32)
    d, nbr_acc = lax.fori_loop(0, K, it, (d, nbr0))
    nbr_ref[0] = nbr_acc + b * N


def _knn(pc, pcT):
    B = pc.shape[0]
    R = _KNN_R
    grid = (B, N // R)
    return pl.pallas_call(
        _knn_body,
        grid=grid,
        in_specs=[
            pl.BlockSpec((1, R, 3), lambda b, i: (b, i, 0)),
            pl.BlockSpec((1, N, 3), lambda b, i: (b, 0, 0)),
            pl.BlockSpec((1, 3, N), lambda b, i: (b, 0, 0)),
        ],
        out_specs=[
            pl.BlockSpec((1, R, K), lambda b, i: (b, i, 0)),
            pl.BlockSpec((1, K, R, 3), lambda b, i: (b, 0, i, 0)),
            pl.BlockSpec((1, K, R, 3), lambda b, i: (b, 0, i, 0)),
        ],
        out_shape=[
            jax.ShapeDtypeStruct((B, N, K), jnp.int32),
            jax.ShapeDtypeStruct((B, K, N, 3), jnp.float32),
            jax.ShapeDtypeStruct((B, K, N, 3), jnp.float32),
        ],
    )(pc, pc, pcT)


_SC_CHUNK = 128


def _sc_gather(table, idx, d_out):
    M = idx.shape[0]
    D = table.shape[1]
    info = plsc.get_sparse_core_info()
    NW = info.num_cores * info.num_subcores
    m_per_w = M // NW
    CH = _SC_CHUNK
    n_iter = m_per_w // CH
    mesh = plsc.VectorSubcoreMesh(core_axis_name="c", subcore_axis_name="s")

    @functools.partial(
        pl.kernel,
        out_type=jax.ShapeDtypeStruct((M, D), jnp.float32),
        mesh=mesh,
        scratch_types=[
            pltpu.VMEM((CH,), jnp.int32),
            pltpu.VMEM((CH, D), jnp.float32),
            pltpu.SemaphoreType.DMA,
        ],
    )
    def gather_kernel(table_hbm, idx_hbm, out_hbm, idx_v, rows_v, sem):
        wid = lax.axis_index("s") * info.num_cores + lax.axis_index("c")
        base = wid * m_per_w

        def body(i, carry):
            off = base + i * CH
            pltpu.sync_copy(idx_hbm.at[pl.ds(off, CH)], idx_v)
            pltpu.async_copy(table_hbm.at[idx_v], rows_v, sem).wait()
            pltpu.sync_copy(rows_v, out_hbm.at[pl.ds(off, CH)])
            return carry

        lax.fori_loop(0, n_iter, body, 0)

    return gather_kernel(table, idx)[:, :d_out]


_CONV_R = 2048
_EPS = 1e-5


def _lrelu(x):
    return jnp.where(x >= 0, x, 0.1 * x)


def _norm(x, stats_blk, g, be):
    m_rows = jnp.float32(NK)
    s1 = stats_blk[0:1, :]
    s2 = stats_blk[1:2, :]
    mean = s1 / m_rows
    var = s2 / m_rows - mean * mean
    inv = lax.rsqrt(var + _EPS)
    return (x - mean) * inv * g + be


def _chain(G, E, ws, stats, upto):
    X = (lax.dot_general(G, ws["W1g"], (((1,), (0,)), ((), ())),
                         preferred_element_type=jnp.float32)
         + lax.dot_general(E, ws["W1e"], (((1,), (0,)), ((), ())),
                           preferred_element_type=jnp.float32)
         + ws["b1"])
    for j in (2, 3):
        if upto < j:
            break
        Xn = _lrelu(_norm(X, stats[j - 2], ws["g%d" % (j - 1)], ws["be%d" % (j - 1)]))
        X = lax.dot_general(Xn, ws["W%d" % j], (((1,), (0,)), ((), ())),
                            preferred_element_type=jnp.float32) + ws["b%d" % j]
    return X


def _w_names(stage):
    names = ["W1g", "W1e", "b1"]
    for j in (2, 3):
        if stage >= j:
            names += ["g%d" % (j - 1), "be%d" % (j - 1), "W%d" % j, "b%d" % j]
    return names


def _stats_pass_body(stage, refs):
    i = pl.program_id(1)
    G_ref, E_ref = refs[0], refs[1]
    pos = 2
    n_stats = stage - 1
    stats = [refs[pos + k][0] for k in range(n_stats)]
    pos += n_stats
    w_refs = refs[pos:-1]
    out_ref = refs[-1]
    names = _w_names(stage)
    ws = {n: w_refs[k][...] for k, n in enumerate(names)}
    X = _chain(G_ref[0], E_ref[0], ws, stats, stage)
    s1 = jnp.sum(X, axis=0, keepdims=True)
    s2 = jnp.sum(X * X, axis=0, keepdims=True)
    new = jnp.concatenate([s1, s2], axis=0)

    @pl.when(i == 0)
    def _():
        out_ref[0] = new

    @pl.when(i > 0)
    def _():
        out_ref[0] = out_ref[0] + new


def _pool_pass_body(refs):
    G_ref, E_ref = refs[0], refs[1]
    stats = [refs[2][0], refs[3][0], refs[4][0]]
    w_refs = refs[5:-1]
    out_ref = refs[-1]
    names = _w_names(3) + ["g3", "be3"]
    ws = {n: w_refs[k][...] for k, n in enumerate(names)}
    X = _chain(G_ref[0], E_ref[0], ws, stats[:2], 3)
    Xn = _lrelu(_norm(X, stats[2], ws["g3"], ws["be3"]))
    c = Xn.shape[-1]
    P = jnp.max(Xn.reshape(_CONV_R // K, K, c), axis=1)
    out_ref[0] = P


def _conv_layer(G, E, p):
    B, _, C = G.shape
    cout = p["W1"].shape[1]
    R = _CONV_R
    grid = (B, NK // R)

    W1g = p["W1"][:C, :]
    W1e = p["W1"][C:, :]

    def wmap(stage, with_g3):
        arrs = [W1g, W1e, p["b1"].reshape(1, cout)]
        for j in (2, 3):
            if stage >= j:
                arrs += [p["g%d" % (j - 1)].reshape(1, cout),
                         p["be%d" % (j - 1)].reshape(1, cout),
                         p["W%d" % j], p["b%d" % j].reshape(1, cout)]
        if with_g3:
            arrs += [p["g3"].reshape(1, cout), p["be3"].reshape(1, cout)]
        return arrs

    def full_spec(a):
        nd = a.ndim
        return pl.BlockSpec(a.shape, lambda b, i, _nd=nd: (0,) * _nd)

    ge_specs = [
        pl.BlockSpec((1, R, C), lambda b, i: (b, i, 0)),
        pl.BlockSpec((1, R, 3), lambda b, i: (b, i, 0)),
    ]
    stat_spec = pl.BlockSpec((1, 2, cout), lambda b, i: (b, 0, 0))

    stats = []
    for stage in (1, 2, 3):
        warrs = wmap(stage, False)
        body = functools.partial(_stats_pass_body, stage)

        def wrapped(*refs, _body=body):
            _body(refs)

        st = pl.pallas_call(
            wrapped,
            grid=grid,
            in_specs=(ge_specs + [stat_spec] * (stage - 1)
                      + [full_spec(a) for a in warrs]),
            out_specs=stat_spec,
            out_shape=jax.ShapeDtypeStruct((B, 2, cout), jnp.float32),
        )(G, E, *stats, *warrs)
        stats.append(st)

    warrs = wmap(3, True)

    def wrapped_pool(*refs):
        _pool_pass_body(refs)

    pooled = pl.pallas_call(
        wrapped_pool,
        grid=grid,
        in_specs=ge_specs + [stat_spec] * 3 + [full_spec(a) for a in warrs],
        out_specs=pl.BlockSpec((1, R // K, cout), lambda b, i: (b, i, 0)),
        out_shape=jax.ShapeDtypeStruct((B, N, cout), jnp.float32),
    )(G, E, *stats, *warrs)
    return pooled


def kernel(pc, params):
    B = pc.shape[0]
    pcT = jnp.swapaxes(pc, 1, 2)
    nbr, xyz_t, ef_t = _knn(pc, pcT)
    edges = nbr.reshape(-1)
    G1 = jnp.transpose(xyz_t, (0, 2, 1, 3)).reshape(B, NK, 3)
    E = jnp.transpose(ef_t, (0, 2, 1, 3)).reshape(B, NK, 3)
    edge_feats = E.reshape(B * NK, 3)

    def padded_table(pooled):
        flat = pooled.reshape(B * N, -1)
        c = flat.shape[1]
        return jnp.pad(flat, ((0, 0), (0, 128 - c))), c

    pooled1 = _conv_layer(G1, E, params["conv1"])
    t1, c1 = padded_table(pooled1)
    G2 = _sc_gather(t1, edges, c1).reshape(B, NK, c1)
    pooled2 = _conv_layer(G2, E, params["conv2"])
    t2, c2 = padded_table(pooled2)
    G3 = _sc_gather(t2, edges, c2).reshape(B, NK, c2)
    pooled3 = _conv_layer(G3, E, params["conv3"])

    x = jnp.swapaxes(pooled3, 1, 2)
    return (x, edges, edge_feats)


# speedup vs baseline: 3.5008x; 2.1324x over previous
import functools

import jax
import jax.numpy as jnp
from jax import lax
from jax.experimental import pallas as pl
from jax.experimental.pallas import tpu as pltpu
from jax.experimental.pallas import tpu_sc as plsc

K = 32
N = 4096
NK = N * K


_KNN_R = 256


def _knn_body(rows_ref, pcT_ref, nbr_ref, d_ref):
    b = pl.program_id(0)
    rows = rows_ref[0]
    pcT = pcT_ref[0]
    R = rows.shape[0]

    r0 = rows[:, 0:1]
    r1 = rows[:, 1:2]
    r2 = rows[:, 2:3]
    sq_r = (r0 * r0 + r1 * r1) + r2 * r2
    c0 = pcT[0:1, :]
    c1 = pcT[1:2, :]
    c2 = pcT[2:3, :]
    sq_c = (c0 * c0 + c1 * c1) + c2 * c2

    mm = lax.dot_general(rows, pcT, (((1,), (0,)), ((), ())),
                         preferred_element_type=jnp.float32)
    d_ref[...] = (sq_r + sq_c) - 2.0 * mm

    NV = N // 128
    lane_f = lax.broadcasted_iota(jnp.int32, (R, 128), 1).astype(jnp.float32)
    iota_k = lax.broadcasted_iota(jnp.int32, (R, K), 1)
    inf = jnp.float32(jnp.inf)
    bigN = jnp.float32(N)

    def it(t, carry):
        jf_prev, nbr_acc = carry
        v_acc = jnp.full((R, 128), inf, jnp.float32)
        k_acc = jnp.zeros((R, 128), jnp.float32)
        for k in range(NV):
            dk = d_ref[:, k * 128:(k + 1) * 128]
            hit = lane_f == (jf_prev - jnp.float32(k * 128))
            dk = jnp.where(hit, inf, dk)
            d_ref[:, k * 128:(k + 1) * 128] = dk
            better = dk < v_acc
            v_acc = jnp.where(better, dk, v_acc)
            k_acc = jnp.where(better, jnp.float32(k), k_acc)
        m = jnp.min(v_acc, axis=1, keepdims=True)
        jf = jnp.min(jnp.where(v_acc == m, k_acc * 128.0 + lane_f, bigN),
                     axis=1, keepdims=True)
        nbr_acc = jnp.where(iota_k == t, jf.astype(jnp.int32), nbr_acc)
        return jf, nbr_acc

    jf0 = jnp.full((R, 1), -1.0, jnp.float32)
    nbr0 = jnp.zeros((R, K), jnp.int32)
    _, nbr_acc = lax.fori_loop(0, K, it, (jf0, nbr0))
    nbr_ref[0] = nbr_acc + b * N


def _knn(pc, pcT):
    B = pc.shape[0]
    R = _KNN_R
    grid = (B, N // R)
    return pl.pallas_call(
        _knn_body,
        grid=grid,
        in_specs=[
            pl.BlockSpec((1, R, 3), lambda b, i: (b, i, 0)),
            pl.BlockSpec((1, 3, N), lambda b, i: (b, 0, 0)),
        ],
        out_specs=pl.BlockSpec((1, R, K), lambda b, i: (b, i, 0)),
        out_shape=jax.ShapeDtypeStruct((B, N, K), jnp.int32),
        scratch_shapes=[pltpu.VMEM((R, N), jnp.float32)],
    )(pc, pcT)


_EP_R = 2048


def _edge_prep_body(xg_ref, ctr_ref, g1_ref, ef_ref):
    xg = xg_ref[0][:, :3]
    ctr = ctr_ref[0]
    rep = jnp.broadcast_to(ctr[:, None, :], (ctr.shape[0], K, 3))
    rep = rep.reshape(xg.shape[0], 3)
    g1_ref[0] = xg
    ef_ref[0] = xg - rep


def _edge_prep(xyzg, pc):
    B = pc.shape[0]
    R = _EP_R
    grid = (B, NK // R)
    return pl.pallas_call(
        _edge_prep_body,
        grid=grid,
        in_specs=[
            pl.BlockSpec((1, R, 128), lambda b, i: (b, i, 0)),
            pl.BlockSpec((1, R // K, 3), lambda b, i: (b, i, 0)),
        ],
        out_specs=[
            pl.BlockSpec((1, R, 3), lambda b, i: (b, i, 0)),
            pl.BlockSpec((1, R, 3), lambda b, i: (b, i, 0)),
        ],
        out_shape=[
            jax.ShapeDtypeStruct((B, NK, 3), jnp.float32),
            jax.ShapeDtypeStruct((B, NK, 3), jnp.float32),
        ],
    )(xyzg, pc)


_SC_CHUNK = 128


def _sc_gather(table, idx):
    M = idx.shape[0]
    D = table.shape[1]
    info = plsc.get_sparse_core_info()
    NW = info.num_cores * info.num_subcores
    m_per_w = M // NW
    CH = _SC_CHUNK
    n_iter = m_per_w // CH
    mesh = plsc.VectorSubcoreMesh(core_axis_name="c", subcore_axis_name="s")

    @functools.partial(
        pl.kernel,
        out_type=jax.ShapeDtypeStruct((M, D), jnp.float32),
        mesh=mesh,
        scratch_types=[
            pltpu.VMEM((CH,), jnp.int32),
            pltpu.VMEM((CH, D), jnp.float32),
            pltpu.SemaphoreType.DMA,
        ],
    )
    def gather_kernel(table_hbm, idx_hbm, out_hbm, idx_v, rows_v, sem):
        wid = lax.axis_index("s") * info.num_cores + lax.axis_index("c")
        base = wid * m_per_w

        def body(i, carry):
            off = base + i * CH
            pltpu.sync_copy(idx_hbm.at[pl.ds(off, CH)], idx_v)
            pltpu.async_copy(table_hbm.at[idx_v], rows_v, sem).wait()
            pltpu.sync_copy(rows_v, out_hbm.at[pl.ds(off, CH)])
            return carry

        lax.fori_loop(0, n_iter, body, 0)

    return gather_kernel(table, idx)


_CONV_R = 2048
_EPS = 1e-5


def _lrelu(x):
    return jnp.where(x >= 0, x, 0.1 * x)


def _norm(x, stats_blk, g, be):
    m_rows = jnp.float32(NK)
    s1 = stats_blk[0:1, :]
    s2 = stats_blk[1:2, :]
    mean = s1 / m_rows
    var = s2 / m_rows - mean * mean
    inv = lax.rsqrt(var + _EPS)
    return (x - mean) * inv * g + be


def _chain(G, E, ws, stats, upto):
    X = (lax.dot_general(G, ws["W1g"], (((1,), (0,)), ((), ())),
                         preferred_element_type=jnp.float32)
         + lax.dot_general(E, ws["W1e"], (((1,), (0,)), ((), ())),
                           preferred_element_type=jnp.float32)
         + ws["b1"])
    for j in (2, 3):
        if upto < j:
            break
        Xn = _lrelu(_norm(X, stats[j - 2], ws["g%d" % (j - 1)], ws["be%d" % (j - 1)]))
        X = lax.dot_general(Xn, ws["W%d" % j], (((1,), (0,)), ((), ())),
                            preferred_element_type=jnp.float32) + ws["b%d" % j]
    return X


def _w_names(stage):
    names = ["W1g", "W1e", "b1"]
    for j in (2, 3):
        if stage >= j:
            names += ["g%d" % (j - 1), "be%d" % (j - 1), "W%d" % j, "b%d" % j]
    return names


def _stats_pass_body(stage, refs):
    i = pl.program_id(1)
    G_ref, E_ref = refs[0], refs[1]
    pos = 2
    n_stats = stage - 1
    stats = [refs[pos + k][0] for k in range(n_stats)]
    pos += n_stats
    w_refs = refs[pos:-1]
    out_ref = refs[-1]
    names = _w_names(stage)
    ws = {n: w_refs[k][...] for k, n in enumerate(names)}
    X = _chain(G_ref[0], E_ref[0], ws, stats, stage)
    s1 = jnp.sum(X, axis=0, keepdims=True)
    s2 = jnp.sum(X * X, axis=0, keepdims=True)
    new = jnp.concatenate([s1, s2], axis=0)

    @pl.when(i == 0)
    def _():
        out_ref[0] = new

    @pl.when(i > 0)
    def _():
        out_ref[0] = out_ref[0] + new


def _pool_pass_body(refs):
    G_ref, E_ref = refs[0], refs[1]
    stats = [refs[2][0], refs[3][0], refs[4][0]]
    w_refs = refs[5:-1]
    out_ref = refs[-1]
    names = _w_names(3) + ["g3", "be3"]
    ws = {n: w_refs[k][...] for k, n in enumerate(names)}
    X = _chain(G_ref[0], E_ref[0], ws, stats[:2], 3)
    Xn = _lrelu(_norm(X, stats[2], ws["g3"], ws["be3"]))
    c = Xn.shape[-1]
    P = jnp.max(Xn.reshape(_CONV_R // K, K, c), axis=1)
    out_ref[0] = P


def _conv_layer(G, E, p):
    B, _, C = G.shape
    cout = p["W1"].shape[1]
    R = _CONV_R
    grid = (B, NK // R)

    W1g = p["W1"][:C, :]
    W1e = p["W1"][C:, :]

    def wmap(stage, with_g3):
        arrs = [W1g, W1e, p["b1"].reshape(1, cout)]
        for j in (2, 3):
            if stage >= j:
                arrs += [p["g%d" % (j - 1)].reshape(1, cout),
                         p["be%d" % (j - 1)].reshape(1, cout),
                         p["W%d" % j], p["b%d" % j].reshape(1, cout)]
        if with_g3:
            arrs += [p["g3"].reshape(1, cout), p["be3"].reshape(1, cout)]
        return arrs

    def full_spec(a):
        nd = a.ndim
        return pl.BlockSpec(a.shape, lambda b, i, _nd=nd: (0,) * _nd)

    ge_specs = [
        pl.BlockSpec((1, R, C), lambda b, i: (b, i, 0)),
        pl.BlockSpec((1, R, 3), lambda b, i: (b, i, 0)),
    ]
    stat_spec = pl.BlockSpec((1, 2, cout), lambda b, i: (b, 0, 0))

    stats = []
    for stage in (1, 2, 3):
        warrs = wmap(stage, False)
        body = functools.partial(_stats_pass_body, stage)

        def wrapped(*refs, _body=body):
            _body(refs)

        st = pl.pallas_call(
            wrapped,
            grid=grid,
            in_specs=(ge_specs + [stat_spec] * (stage - 1)
                      + [full_spec(a) for a in warrs]),
            out_specs=stat_spec,
            out_shape=jax.ShapeDtypeStruct((B, 2, cout), jnp.float32),
        )(G, E, *stats, *warrs)
        stats.append(st)

    warrs = wmap(3, True)

    def wrapped_pool(*refs):
        _pool_pass_body(refs)

    pooled = pl.pallas_call(
        wrapped_pool,
        grid=grid,
        in_specs=ge_specs + [stat_spec] * 3 + [full_spec(a) for a in warrs],
        out_specs=pl.BlockSpec((1, R // K, cout), lambda b, i: (b, i, 0)),
        out_shape=jax.ShapeDtypeStruct((B, N, cout), jnp.float32),
    )(G, E, *stats, *warrs)
    return pooled


def kernel(pc, params):
    B = pc.shape[0]
    pcT = jnp.swapaxes(pc, 1, 2)
    nbr = _knn(pc, pcT)
    edges = nbr.reshape(-1)
    pc_pad = jnp.pad(pc.reshape(B * N, 3), ((0, 0), (0, 125)))
    xyzg = _sc_gather(pc_pad, edges).reshape(B, NK, 128)
    G1, E = _edge_prep(xyzg, pc)
    edge_feats = E.reshape(B * NK, 3)

    def padded_table(pooled):
        flat = pooled.reshape(B * N, -1)
        c = flat.shape[1]
        return jnp.pad(flat, ((0, 0), (0, 128 - c))), c

    pooled1 = _conv_layer(G1, E, params["conv1"])
    t1, c1 = padded_table(pooled1)
    G2 = _sc_gather(t1, edges)[:, :c1].reshape(B, NK, c1)
    pooled2 = _conv_layer(G2, E, params["conv2"])
    t2, c2 = padded_table(pooled2)
    G3 = _sc_gather(t2, edges)[:, :c2].reshape(B, NK, c2)
    pooled3 = _conv_layer(G3, E, params["conv3"])

    x = jnp.swapaxes(pooled3, 1, 2)
    return (x, edges, edge_feats)


# speedup vs baseline: 3.6839x; 1.0523x over previous
import functools

import jax
import jax.numpy as jnp
from jax import lax
from jax.experimental import pallas as pl
from jax.experimental.pallas import tpu as pltpu
from jax.experimental.pallas import tpu_sc as plsc

K = 32
N = 4096
NK = N * K


_KNN_R = 256


def _knn_body(rows_ref, pcT_ref, nbr_ref, d_ref):
    b = pl.program_id(0)
    rows = rows_ref[0]
    pcT = pcT_ref[0]
    R = rows.shape[0]

    r0 = rows[:, 0:1]
    r1 = rows[:, 1:2]
    r2 = rows[:, 2:3]
    sq_r = (r0 * r0 + r1 * r1) + r2 * r2
    c0 = pcT[0:1, :]
    c1 = pcT[1:2, :]
    c2 = pcT[2:3, :]
    sq_c = (c0 * c0 + c1 * c1) + c2 * c2

    mm = lax.dot_general(rows, pcT, (((1,), (0,)), ((), ())),
                         preferred_element_type=jnp.float32)
    d_ref[...] = (sq_r + sq_c) - 2.0 * mm

    NV = N // 128
    lane_f = lax.broadcasted_iota(jnp.int32, (R, 128), 1).astype(jnp.float32)
    iota_k = lax.broadcasted_iota(jnp.int32, (R, K), 1)
    inf = jnp.float32(jnp.inf)
    bigN = jnp.float32(N)

    def it(t, carry):
        jf_prev, nbr_acc = carry
        v_acc = jnp.full((R, 128), inf, jnp.float32)
        k_acc = jnp.zeros((R, 128), jnp.float32)
        for k in range(NV):
            dk = d_ref[:, k * 128:(k + 1) * 128]
            hit = lane_f == (jf_prev - jnp.float32(k * 128))
            dk = jnp.where(hit, inf, dk)
            d_ref[:, k * 128:(k + 1) * 128] = dk
            better = dk < v_acc
            v_acc = jnp.where(better, dk, v_acc)
            k_acc = jnp.where(better, jnp.float32(k), k_acc)
        m = jnp.min(v_acc, axis=1, keepdims=True)
        jf = jnp.min(jnp.where(v_acc == m, k_acc * 128.0 + lane_f, bigN),
                     axis=1, keepdims=True)
        nbr_acc = jnp.where(iota_k == t, jf.astype(jnp.int32), nbr_acc)
        return jf, nbr_acc

    jf0 = jnp.full((R, 1), -1.0, jnp.float32)
    nbr0 = jnp.zeros((R, K), jnp.int32)
    _, nbr_acc = lax.fori_loop(0, K, it, (jf0, nbr0))
    nbr_ref[0] = nbr_acc + b * N


def _knn(pc, pcT):
    B = pc.shape[0]
    R = _KNN_R
    grid = (B, N // R)
    return pl.pallas_call(
        _knn_body,
        grid=grid,
        in_specs=[
            pl.BlockSpec((1, R, 3), lambda b, i: (b, i, 0)),
            pl.BlockSpec((1, 3, N), lambda b, i: (b, 0, 0)),
        ],
        out_specs=pl.BlockSpec((1, R, K), lambda b, i: (b, i, 0)),
        out_shape=jax.ShapeDtypeStruct((B, N, K), jnp.int32),
        scratch_shapes=[pltpu.VMEM((R, N), jnp.float32)],
    )(pc, pcT)


_EP_R = 2048


def _edge_prep_body(xg_ref, ctr_ref, g1_ref, ef_ref):
    xg = xg_ref[0][:, :3]
    ctr = ctr_ref[0]
    rep = jnp.broadcast_to(ctr[:, None, :], (ctr.shape[0], K, 3))
    rep = rep.reshape(xg.shape[0], 3)
    g1_ref[0] = xg
    ef_ref[0] = xg - rep


def _edge_prep(xyzg, pc):
    B = pc.shape[0]
    R = _EP_R
    grid = (B, NK // R)
    return pl.pallas_call(
        _edge_prep_body,
        grid=grid,
        in_specs=[
            pl.BlockSpec((1, R, 128), lambda b, i: (b, i, 0)),
            pl.BlockSpec((1, R // K, 3), lambda b, i: (b, i, 0)),
        ],
        out_specs=[
            pl.BlockSpec((1, R, 3), lambda b, i: (b, i, 0)),
            pl.BlockSpec((1, R, 3), lambda b, i: (b, i, 0)),
        ],
        out_shape=[
            jax.ShapeDtypeStruct((B, NK, 3), jnp.float32),
            jax.ShapeDtypeStruct((B, NK, 3), jnp.float32),
        ],
    )(xyzg, pc)


_SC_CHUNK = 128
_SC_NBUF = 4


def _sc_gather(table, idx):
    M = idx.shape[0]
    D = table.shape[1]
    info = plsc.get_sparse_core_info()
    NW = info.num_cores * info.num_subcores
    m_per_w = M // NW
    CH = _SC_CHUNK
    NB = _SC_NBUF
    n_iter = m_per_w // CH
    mesh = plsc.VectorSubcoreMesh(core_axis_name="c", subcore_axis_name="s")

    @functools.partial(
        pl.kernel,
        out_type=jax.ShapeDtypeStruct((M, D), jnp.float32),
        mesh=mesh,
        scratch_types=[
            pltpu.VMEM((NB, CH), jnp.int32),
            pltpu.VMEM((NB, CH, D), jnp.float32),
            [pltpu.SemaphoreType.DMA] * NB,
        ],
    )
    def gather_kernel(table_hbm, idx_hbm, out_hbm, idx_v, rows_v, sems):
        wid = lax.axis_index("s") * info.num_cores + lax.axis_index("c")
        base = wid * m_per_w
        copies = [None] * NB

        def start(i, buf):
            off = base + i * CH
            pltpu.sync_copy(idx_hbm.at[pl.ds(off, CH)], idx_v.at[buf])
            copies[buf] = pltpu.async_copy(
                table_hbm.at[idx_v.at[buf]], rows_v.at[buf], sems[buf])

        for i in range(min(NB, n_iter)):
            start(i, i)
        for i in range(n_iter):
            buf = i % NB
            copies[buf].wait()
            pltpu.sync_copy(rows_v.at[buf], out_hbm.at[pl.ds(base + i * CH, CH)])
            if i + NB < n_iter:
                start(i + NB, buf)

    return gather_kernel(table, idx)


_CONV_R = 2048
_EPS = 1e-5


def _lrelu(x):
    return jnp.where(x >= 0, x, 0.1 * x)


def _norm(x, stats_blk, g, be):
    m_rows = jnp.float32(NK)
    s1 = stats_blk[0:1, :]
    s2 = stats_blk[1:2, :]
    mean = s1 / m_rows
    var = s2 / m_rows - mean * mean
    inv = lax.rsqrt(var + _EPS)
    return (x - mean) * inv * g + be


def _chain(G, E, ws, stats, upto):
    X = (lax.dot_general(G, ws["W1g"], (((1,), (0,)), ((), ())),
                         preferred_element_type=jnp.float32)
         + lax.dot_general(E, ws["W1e"], (((1,), (0,)), ((), ())),
                           preferred_element_type=jnp.float32)
         + ws["b1"])
    for j in (2, 3):
        if upto < j:
            break
        Xn = _lrelu(_norm(X, stats[j - 2], ws["g%d" % (j - 1)], ws["be%d" % (j - 1)]))
        X = lax.dot_general(Xn, ws["W%d" % j], (((1,), (0,)), ((), ())),
                            preferred_element_type=jnp.float32) + ws["b%d" % j]
    return X


def _w_names(stage):
    names = ["W1g", "W1e", "b1"]
    for j in (2, 3):
        if stage >= j:
            names += ["g%d" % (j - 1), "be%d" % (j - 1), "W%d" % j, "b%d" % j]
    return names


def _stats_pass_body(stage, refs):
    i = pl.program_id(1)
    G_ref, E_ref = refs[0], refs[1]
    pos = 2
    n_stats = stage - 1
    stats = [refs[pos + k][0] for k in range(n_stats)]
    pos += n_stats
    w_refs = refs[pos:-1]
    out_ref = refs[-1]
    names = _w_names(stage)
    ws = {n: w_refs[k][...] for k, n in enumerate(names)}
    X = _chain(G_ref[0], E_ref[0], ws, stats, stage)
    s1 = jnp.sum(X, axis=0, keepdims=True)
    s2 = jnp.sum(X * X, axis=0, keepdims=True)
    new = jnp.concatenate([s1, s2], axis=0)

    @pl.when(i == 0)
    def _():
        out_ref[0] = new

    @pl.when(i > 0)
    def _():
        out_ref[0] = out_ref[0] + new


def _pool_pass_body(refs):
    G_ref, E_ref = refs[0], refs[1]
    stats = [refs[2][0], refs[3][0], refs[4][0]]
    w_refs = refs[5:-1]
    out_ref = refs[-1]
    names = _w_names(3) + ["g3", "be3"]
    ws = {n: w_refs[k][...] for k, n in enumerate(names)}
    X = _chain(G_ref[0], E_ref[0], ws, stats[:2], 3)
    Xn = _lrelu(_norm(X, stats[2], ws["g3"], ws["be3"]))
    c = Xn.shape[-1]
    P = jnp.max(Xn.reshape(_CONV_R // K, K, c), axis=1)
    out_ref[0] = P


def _conv_layer(G, E, p):
    B, _, C = G.shape
    cout = p["W1"].shape[1]
    R = _CONV_R
    grid = (B, NK // R)

    W1g = p["W1"][:C, :]
    W1e = p["W1"][C:, :]

    def wmap(stage, with_g3):
        arrs = [W1g, W1e, p["b1"].reshape(1, cout)]
        for j in (2, 3):
            if stage >= j:
                arrs += [p["g%d" % (j - 1)].reshape(1, cout),
                         p["be%d" % (j - 1)].reshape(1, cout),
                         p["W%d" % j], p["b%d" % j].reshape(1, cout)]
        if with_g3:
            arrs += [p["g3"].reshape(1, cout), p["be3"].reshape(1, cout)]
        return arrs

    def full_spec(a):
        nd = a.ndim
        return pl.BlockSpec(a.shape, lambda b, i, _nd=nd: (0,) * _nd)

    ge_specs = [
        pl.BlockSpec((1, R, C), lambda b, i: (b, i, 0)),
        pl.BlockSpec((1, R, 3), lambda b, i: (b, i, 0)),
    ]
    stat_spec = pl.BlockSpec((1, 2, cout), lambda b, i: (b, 0, 0))

    stats = []
    for stage in (1, 2, 3):
        warrs = wmap(stage, False)
        body = functools.partial(_stats_pass_body, stage)

        def wrapped(*refs, _body=body):
            _body(refs)

        st = pl.pallas_call(
            wrapped,
            grid=grid,
            in_specs=(ge_specs + [stat_spec] * (stage - 1)
                      + [full_spec(a) for a in warrs]),
            out_specs=stat_spec,
            out_shape=jax.ShapeDtypeStruct((B, 2, cout), jnp.float32),
        )(G, E, *stats, *warrs)
        stats.append(st)

    warrs = wmap(3, True)

    def wrapped_pool(*refs):
        _pool_pass_body(refs)

    pooled = pl.pallas_call(
        wrapped_pool,
        grid=grid,
        in_specs=ge_specs + [stat_spec] * 3 + [full_spec(a) for a in warrs],
        out_specs=pl.BlockSpec((1, R // K, cout), lambda b, i: (b, i, 0)),
        out_shape=jax.ShapeDtypeStruct((B, N, cout), jnp.float32),
    )(G, E, *stats, *warrs)
    return pooled


def kernel(pc, params):
    B = pc.shape[0]
    pcT = jnp.swapaxes(pc, 1, 2)
    nbr = _knn(pc, pcT)
    edges = nbr.reshape(-1)
    pc_pad = jnp.pad(pc.reshape(B * N, 3), ((0, 0), (0, 125)))
    xyzg = _sc_gather(pc_pad, edges).reshape(B, NK, 128)
    G1, E = _edge_prep(xyzg, pc)
    edge_feats = E.reshape(B * NK, 3)

    def padded_table(pooled):
        flat = pooled.reshape(B * N, -1)
        c = flat.shape[1]
        return jnp.pad(flat, ((0, 0), (0, 128 - c))), c

    pooled1 = _conv_layer(G1, E, params["conv1"])
    t1, c1 = padded_table(pooled1)
    G2 = _sc_gather(t1, edges)[:, :c1].reshape(B, NK, c1)
    pooled2 = _conv_layer(G2, E, params["conv2"])
    t2, c2 = padded_table(pooled2)
    G3 = _sc_gather(t2, edges)[:, :c2].reshape(B, NK, c2)
    pooled3 = _conv_layer(G3, E, params["conv3"])

    x = jnp.swapaxes(pooled3, 1, 2)
    return (x, edges, edge_feats)


# speedup vs baseline: 3.8134x; 1.0351x over previous
import functools

import jax
import jax.numpy as jnp
from jax import lax
from jax.experimental import pallas as pl
from jax.experimental.pallas import tpu as pltpu
from jax.experimental.pallas import tpu_sc as plsc

K = 32
N = 4096
NK = N * K


_KNN_R = 256


def _knn_body(rows_ref, pcT_ref, nbr_ref, d_ref):
    b = pl.program_id(0)
    rows = rows_ref[0]
    pcT = pcT_ref[0]
    R = rows.shape[0]

    r0 = rows[:, 0:1]
    r1 = rows[:, 1:2]
    r2 = rows[:, 2:3]
    sq_r = (r0 * r0 + r1 * r1) + r2 * r2
    c0 = pcT[0:1, :]
    c1 = pcT[1:2, :]
    c2 = pcT[2:3, :]
    sq_c = (c0 * c0 + c1 * c1) + c2 * c2

    mm = lax.dot_general(rows, pcT, (((1,), (0,)), ((), ())),
                         preferred_element_type=jnp.float32)
    d_ref[...] = (sq_r + sq_c) - 2.0 * mm

    NV = N // 128
    lane_f = lax.broadcasted_iota(jnp.int32, (R, 128), 1).astype(jnp.float32)
    iota_k = lax.broadcasted_iota(jnp.int32, (R, K), 1)
    inf = jnp.float32(jnp.inf)
    bigN = jnp.float32(N)

    def it(t, carry):
        jf_prev, nbr_acc = carry
        v_acc = jnp.full((R, 128), inf, jnp.float32)
        k_acc = jnp.zeros((R, 128), jnp.float32)
        for k in range(NV):
            dk = d_ref[:, k * 128:(k + 1) * 128]
            hit = lane_f == (jf_prev - jnp.float32(k * 128))
            dk = jnp.where(hit, inf, dk)
            d_ref[:, k * 128:(k + 1) * 128] = dk
            better = dk < v_acc
            v_acc = jnp.where(better, dk, v_acc)
            k_acc = jnp.where(better, jnp.float32(k), k_acc)
        m = jnp.min(v_acc, axis=1, keepdims=True)
        jf = jnp.min(jnp.where(v_acc == m, k_acc * 128.0 + lane_f, bigN),
                     axis=1, keepdims=True)
        nbr_acc = jnp.where(iota_k == t, jf.astype(jnp.int32), nbr_acc)
        return jf, nbr_acc

    jf0 = jnp.full((R, 1), -1.0, jnp.float32)
    nbr0 = jnp.zeros((R, K), jnp.int32)
    _, nbr_acc = lax.fori_loop(0, K, it, (jf0, nbr0))
    nbr_ref[0] = nbr_acc + b * N


def _knn(pc, pcT):
    B = pc.shape[0]
    R = _KNN_R
    grid = (B, N // R)
    return pl.pallas_call(
        _knn_body,
        grid=grid,
        in_specs=[
            pl.BlockSpec((1, R, 3), lambda b, i: (b, i, 0)),
            pl.BlockSpec((1, 3, N), lambda b, i: (b, 0, 0)),
        ],
        out_specs=pl.BlockSpec((1, R, K), lambda b, i: (b, i, 0)),
        out_shape=jax.ShapeDtypeStruct((B, N, K), jnp.int32),
        scratch_shapes=[pltpu.VMEM((R, N), jnp.float32)],
    )(pc, pcT)


_EP_R = 2048


def _edge_prep_body(xg_ref, ctr_ref, ge_ref, ef_ref):
    xg = xg_ref[0][:, :3]
    ctr = ctr_ref[0]
    rep = jnp.broadcast_to(ctr[:, None, :], (ctr.shape[0], K, 3))
    rep = rep.reshape(xg.shape[0], 3)
    ef = xg - rep
    zero2 = jnp.zeros((xg.shape[0], 2), jnp.float32)
    ge_ref[0] = jnp.concatenate([xg, ef, zero2], axis=1)
    ef_ref[0] = ef


def _edge_prep(xyzg, pc):
    B = pc.shape[0]
    R = _EP_R
    grid = (B, NK // R)
    return pl.pallas_call(
        _edge_prep_body,
        grid=grid,
        in_specs=[
            pl.BlockSpec((1, R, 128), lambda b, i: (b, i, 0)),
            pl.BlockSpec((1, R // K, 3), lambda b, i: (b, i, 0)),
        ],
        out_specs=[
            pl.BlockSpec((1, R, 8), lambda b, i: (b, i, 0)),
            pl.BlockSpec((1, R, 3), lambda b, i: (b, i, 0)),
        ],
        out_shape=[
            jax.ShapeDtypeStruct((B, NK, 8), jnp.float32),
            jax.ShapeDtypeStruct((B, NK, 3), jnp.float32),
        ],
    )(xyzg, pc)


_SC_CHUNK = 128
_SC_NBUF = 4


def _sc_gather(table, idx):
    M = idx.shape[0]
    D = table.shape[1]
    info = plsc.get_sparse_core_info()
    NW = info.num_cores * info.num_subcores
    m_per_w = M // NW
    CH = _SC_CHUNK
    NB = _SC_NBUF
    n_iter = m_per_w // CH
    mesh = plsc.VectorSubcoreMesh(core_axis_name="c", subcore_axis_name="s")

    @functools.partial(
        pl.kernel,
        out_type=jax.ShapeDtypeStruct((M, D), jnp.float32),
        mesh=mesh,
        scratch_types=[
            pltpu.VMEM((NB, CH), jnp.int32),
            pltpu.VMEM((NB, CH, D), jnp.float32),
            [pltpu.SemaphoreType.DMA] * NB,
        ],
    )
    def gather_kernel(table_hbm, idx_hbm, out_hbm, idx_v, rows_v, sems):
        wid = lax.axis_index("s") * info.num_cores + lax.axis_index("c")
        base = wid * m_per_w
        copies = [None] * NB

        def start(i, buf):
            off = base + i * CH
            pltpu.sync_copy(idx_hbm.at[pl.ds(off, CH)], idx_v.at[buf])
            copies[buf] = pltpu.async_copy(
                table_hbm.at[idx_v.at[buf]], rows_v.at[buf], sems[buf])

        for i in range(min(NB, n_iter)):
            start(i, i)
        for i in range(n_iter):
            buf = i % NB
            copies[buf].wait()
            pltpu.sync_copy(rows_v.at[buf], out_hbm.at[pl.ds(base + i * CH, CH)])
            if i + NB < n_iter:
                start(i + NB, buf)

    return gather_kernel(table, idx)


_CONV_R = 2048
_EPS = 1e-5


def _lrelu(x):
    return jnp.where(x >= 0, x, 0.1 * x)


def _norm(x, stats_blk, g, be):
    m_rows = jnp.float32(NK)
    s1 = stats_blk[0:1, :]
    s2 = stats_blk[1:2, :]
    mean = s1 / m_rows
    var = s2 / m_rows - mean * mean
    inv = lax.rsqrt(var + _EPS)
    return (x - mean) * inv * g + be


def _dot(a, w):
    return lax.dot_general(a, w, (((1,), (0,)), ((), ())),
                           preferred_element_type=jnp.float32)


def _chain(ins, ws, stats, upto):
    X = sum(_dot(a, w) for a, w in ins) + ws["b1"]
    for j in (2, 3):
        if upto < j:
            break
        Xn = _lrelu(_norm(X, stats[j - 2], ws["g%d" % (j - 1)], ws["be%d" % (j - 1)]))
        X = _dot(Xn, ws["W%d" % j]) + ws["b%d" % j]
    return X


def _w_names(stage):
    names = ["b1"]
    for j in (2, 3):
        if stage >= j:
            names += ["g%d" % (j - 1), "be%d" % (j - 1), "W%d" % j, "b%d" % j]
    return names


def _pass_body(stage, n_in, pool, refs):
    i = pl.program_id(1)
    ins = [(refs[2 * k][0], refs[2 * k + 1][...]) for k in range(n_in)]
    pos = 2 * n_in
    n_stats = stage - 1 + (1 if pool else 0)
    stats = [refs[pos + k][0] for k in range(n_stats)]
    pos += n_stats
    w_refs = refs[pos:-1]
    out_ref = refs[-1]
    names = _w_names(stage) + (["g3", "be3"] if pool else [])
    ws = {n: w_refs[k][...] for k, n in enumerate(names)}
    X = _chain(ins, ws, stats, stage)
    if pool:
        Xn = _lrelu(_norm(X, stats[-1], ws["g3"], ws["be3"]))
        c = Xn.shape[-1]
        P = jnp.max(Xn.reshape(Xn.shape[0] // K, K, c), axis=1)
        out_ref[0] = P
    else:
        s1 = jnp.sum(X, axis=0, keepdims=True)
        s2 = jnp.sum(X * X, axis=0, keepdims=True)
        new = jnp.concatenate([s1, s2], axis=0)

        @pl.when(i == 0)
        def _():
            out_ref[0] = new

        @pl.when(i > 0)
        def _():
            out_ref[0] = out_ref[0] + new


def _conv_layer(inputs, p, cout):
    B = inputs[0][0].shape[0]
    R = _CONV_R
    grid = (B, NK // R)
    n_in = len(inputs)

    def wmap(stage, with_g3):
        arrs = [p["b1"].reshape(1, cout)]
        for j in (2, 3):
            if stage >= j:
                arrs += [p["g%d" % (j - 1)].reshape(1, cout),
                         p["be%d" % (j - 1)].reshape(1, cout),
                         p["W%d" % j], p["b%d" % j].reshape(1, cout)]
        if with_g3:
            arrs += [p["g3"].reshape(1, cout), p["be3"].reshape(1, cout)]
        return arrs

    def full_spec(a):
        nd = a.ndim
        return pl.BlockSpec(a.shape, lambda b, i, _nd=nd: (0,) * _nd)

    in_arrs = []
    in_specs = []
    for a, w in inputs:
        in_arrs += [a, w]
        in_specs += [pl.BlockSpec((1, R, a.shape[2]), lambda b, i: (b, i, 0)),
                     full_spec(w)]
    stat_spec = pl.BlockSpec((1, 2, cout), lambda b, i: (b, 0, 0))

    stats = []
    for stage in (1, 2, 3, 4):
        pool = stage == 4
        s = min(stage, 3)
        warrs = wmap(s, pool)
        body = functools.partial(_pass_body, s, n_in, pool)

        def wrapped(*refs, _body=body):
            _body(refs)

        n_stats = s - 1 + (1 if pool else 0)
        if pool:
            out_spec = pl.BlockSpec((1, R // K, cout), lambda b, i: (b, i, 0))
            out_shape = jax.ShapeDtypeStruct((B, N, cout), jnp.float32)
        else:
            out_spec = stat_spec
            out_shape = jax.ShapeDtypeStruct((B, 2, cout), jnp.float32)
        res = pl.pallas_call(
            wrapped,
            grid=grid,
            in_specs=(in_specs + [stat_spec] * n_stats
                      + [full_spec(a) for a in warrs]),
            out_specs=out_spec,
            out_shape=out_shape,
        )(*in_arrs, *stats[:n_stats], *warrs)
        if pool:
            return res
        stats.append(res)


def kernel(pc, params):
    B = pc.shape[0]
    pcT = jnp.swapaxes(pc, 1, 2)
    nbr = _knn(pc, pcT)
    edges = nbr.reshape(-1)
    pc_pad = jnp.pad(pc.reshape(B * N, 3), ((0, 0), (0, 125)))
    xyzg = _sc_gather(pc_pad, edges).reshape(B, NK, 128)
    GE8, E3 = _edge_prep(xyzg, pc)
    edge_feats = E3.reshape(B * NK, 3)

    def we8(W, c_in):
        z = jnp.zeros((8, W.shape[1]), jnp.float32)
        return z.at[3:6].set(W[c_in:])

    W1 = params["conv1"]["W1"]
    Wc1 = jnp.pad(W1, ((0, 2), (0, 0)))
    pooled1 = _conv_layer([(GE8, Wc1)], params["conv1"], 32)

    t1 = jnp.pad(pooled1.reshape(B * N, 32), ((0, 0), (0, 96)))
    G2 = _sc_gather(t1, edges).reshape(B, NK, 128)
    W2 = params["conv2"]["W1"]
    pooled2 = _conv_layer(
        [(G2, jnp.pad(W2[:32], ((0, 96), (0, 0)))), (GE8, we8(W2, 32))],
        params["conv2"], 64)

    t2 = jnp.pad(pooled2.reshape(B * N, 64), ((0, 0), (0, 64)))
    G3 = _sc_gather(t2, edges).reshape(B, NK, 128)
    W3 = params["conv3"]["W1"]
    pooled3 = _conv_layer(
        [(G3, jnp.pad(W3[:64], ((0, 64), (0, 0)))), (GE8, we8(W3, 64))],
        params["conv3"], 128)

    x = jnp.swapaxes(pooled3, 1, 2)
    return (x, edges, edge_feats)


# speedup vs baseline: 3.8938x; 1.0211x over previous
import functools

import jax
import jax.numpy as jnp
from jax import lax
from jax.experimental import pallas as pl
from jax.experimental.pallas import tpu as pltpu
from jax.experimental.pallas import tpu_sc as plsc

K = 32
N = 4096
NK = N * K


_KNN_R = 256


def _knn_body(rows_ref, pcT_ref, nbr_ref, d_ref):
    b = pl.program_id(0)
    rows = rows_ref[0]
    pcT = pcT_ref[0]
    R = rows.shape[0]

    r0 = rows[:, 0:1]
    r1 = rows[:, 1:2]
    r2 = rows[:, 2:3]
    sq_r = (r0 * r0 + r1 * r1) + r2 * r2
    c0 = pcT[0:1, :]
    c1 = pcT[1:2, :]
    c2 = pcT[2:3, :]
    sq_c = (c0 * c0 + c1 * c1) + c2 * c2

    mm = lax.dot_general(rows, pcT, (((1,), (0,)), ((), ())),
                         preferred_element_type=jnp.float32)
    d_ref[...] = (sq_r + sq_c) - 2.0 * mm

    NV = N // 128
    lane_f = lax.broadcasted_iota(jnp.int32, (R, 128), 1).astype(jnp.float32)
    iota_k = lax.broadcasted_iota(jnp.int32, (R, K), 1)
    inf = jnp.float32(jnp.inf)
    bigN = jnp.float32(N)

    def it(t, carry):
        jf_prev, nbr_acc = carry
        v_acc = jnp.full((R, 128), inf, jnp.float32)
        k_acc = jnp.zeros((R, 128), jnp.float32)
        for k in range(NV):
            dk = d_ref[:, k * 128:(k + 1) * 128]
            hit = lane_f == (jf_prev - jnp.float32(k * 128))
            dk = jnp.where(hit, inf, dk)
            d_ref[:, k * 128:(k + 1) * 128] = dk
            better = dk < v_acc
            v_acc = jnp.where(better, dk, v_acc)
            k_acc = jnp.where(better, jnp.float32(k), k_acc)
        m = jnp.min(v_acc, axis=1, keepdims=True)
        jf = jnp.min(jnp.where(v_acc == m, k_acc * 128.0 + lane_f, bigN),
                     axis=1, keepdims=True)
        nbr_acc = jnp.where(iota_k == t, jf.astype(jnp.int32), nbr_acc)
        return jf, nbr_acc

    jf0 = jnp.full((R, 1), -1.0, jnp.float32)
    nbr0 = jnp.zeros((R, K), jnp.int32)
    _, nbr_acc = lax.fori_loop(0, K, it, (jf0, nbr0))
    nbr_ref[0] = nbr_acc + b * N


def _knn(pc, pcT):
    B = pc.shape[0]
    R = _KNN_R
    grid = (B, N // R)
    return pl.pallas_call(
        _knn_body,
        grid=grid,
        in_specs=[
            pl.BlockSpec((1, R, 3), lambda b, i: (b, i, 0)),
            pl.BlockSpec((1, 3, N), lambda b, i: (b, 0, 0)),
        ],
        out_specs=pl.BlockSpec((1, R, K), lambda b, i: (b, i, 0)),
        out_shape=jax.ShapeDtypeStruct((B, N, K), jnp.int32),
        scratch_shapes=[pltpu.VMEM((R, N), jnp.float32)],
    )(pc, pcT)


_EP_R = 2048


def _edge_prep_body(xg_ref, ctr_ref, ge_ref, ef_ref):
    xg = xg_ref[0][:, :3]
    ctr = ctr_ref[0]
    rep = jnp.broadcast_to(ctr[:, None, :], (ctr.shape[0], K, 3))
    rep = rep.reshape(xg.shape[0], 3)
    ef = xg - rep
    zero2 = jnp.zeros((xg.shape[0], 2), jnp.float32)
    ge_ref[0] = jnp.concatenate([xg, ef, zero2], axis=1)
    ef_ref[0] = ef


def _edge_prep(xyzg, pc):
    B = pc.shape[0]
    R = _EP_R
    grid = (B, NK // R)
    return pl.pallas_call(
        _edge_prep_body,
        grid=grid,
        in_specs=[
            pl.BlockSpec((1, R, 128), lambda b, i: (b, i, 0)),
            pl.BlockSpec((1, R // K, 3), lambda b, i: (b, i, 0)),
        ],
        out_specs=[
            pl.BlockSpec((1, R, 8), lambda b, i: (b, i, 0)),
            pl.BlockSpec((1, R, 3), lambda b, i: (b, i, 0)),
        ],
        out_shape=[
            jax.ShapeDtypeStruct((B, NK, 8), jnp.float32),
            jax.ShapeDtypeStruct((B, NK, 3), jnp.float32),
        ],
    )(xyzg, pc)


_SC_CHUNK = 128
_SC_NBUF = 4


def _sc_gather(table, idx):
    M = idx.shape[0]
    D = table.shape[1]
    info = plsc.get_sparse_core_info()
    NW = info.num_cores * info.num_subcores
    m_per_w = M // NW
    CH = _SC_CHUNK
    NB = _SC_NBUF
    n_iter = m_per_w // CH
    mesh = plsc.VectorSubcoreMesh(core_axis_name="c", subcore_axis_name="s")

    @functools.partial(
        pl.kernel,
        out_type=jax.ShapeDtypeStruct((M, D), jnp.float32),
        mesh=mesh,
        scratch_types=[
            pltpu.VMEM((NB, CH), jnp.int32),
            pltpu.VMEM((NB, CH, D), jnp.float32),
            [pltpu.SemaphoreType.DMA] * NB,
        ],
    )
    def gather_kernel(table_hbm, idx_hbm, out_hbm, idx_v, rows_v, sems):
        wid = lax.axis_index("s") * info.num_cores + lax.axis_index("c")
        base = wid * m_per_w
        copies = [None] * NB

        def start(i, buf):
            off = base + i * CH
            pltpu.sync_copy(idx_hbm.at[pl.ds(off, CH)], idx_v.at[buf])
            copies[buf] = pltpu.async_copy(
                table_hbm.at[idx_v.at[buf]], rows_v.at[buf], sems[buf])

        for i in range(min(NB, n_iter)):
            start(i, i)
        for i in range(n_iter):
            buf = i % NB
            copies[buf].wait()
            pltpu.sync_copy(rows_v.at[buf], out_hbm.at[pl.ds(base + i * CH, CH)])
            if i + NB < n_iter:
                start(i + NB, buf)

    return gather_kernel(table, idx)


_CONV_R = 2048
_EPS = 1e-5


def _lrelu(x):
    return jnp.where(x >= 0, x, 0.1 * x)


def _norm(x, stats_blk, g, be):
    m_rows = jnp.float32(NK)
    s1 = stats_blk[0:1, :]
    s2 = stats_blk[1:2, :]
    mean = s1 / m_rows
    var = s2 / m_rows - mean * mean
    inv = lax.rsqrt(var + _EPS)
    return (x - mean) * inv * g + be


def _dot(a, w, is_t):
    if is_t:
        return lax.dot_general(a, w, (((0,), (0,)), ((), ())),
                               preferred_element_type=jnp.float32)
    return lax.dot_general(a, w, (((1,), (0,)), ((), ())),
                           preferred_element_type=jnp.float32)


def _chain(ins, ws, stats, upto):
    X = sum(_dot(a, w, t) for a, w, t in ins) + ws["b1"]
    for j in (2, 3):
        if upto < j:
            break
        Xn = _lrelu(_norm(X, stats[j - 2], ws["g%d" % (j - 1)], ws["be%d" % (j - 1)]))
        X = _dot(Xn, ws["W%d" % j], False) + ws["b%d" % j]
    return X


def _w_names(stage):
    names = ["b1"]
    for j in (2, 3):
        if stage >= j:
            names += ["g%d" % (j - 1), "be%d" % (j - 1), "W%d" % j, "b%d" % j]
    return names


def _pass_body(stage, t_flags, pool, refs):
    i = pl.program_id(1)
    n_in = len(t_flags)
    ins = [(refs[2 * k][0], refs[2 * k + 1][...], t_flags[k])
           for k in range(n_in)]
    pos = 2 * n_in
    n_stats = stage - 1 + (1 if pool else 0)
    stats = [refs[pos + k][0] for k in range(n_stats)]
    pos += n_stats
    w_refs = refs[pos:-1]
    out_ref = refs[-1]
    names = _w_names(stage) + (["g3", "be3"] if pool else [])
    ws = {n: w_refs[k][...] for k, n in enumerate(names)}
    X = _chain(ins, ws, stats, stage)
    if pool:
        Xn = _lrelu(_norm(X, stats[-1], ws["g3"], ws["be3"]))
        c = Xn.shape[-1]
        P = jnp.max(Xn.reshape(Xn.shape[0] // K, K, c), axis=1)
        out_ref[0] = P
    else:
        s1 = jnp.sum(X, axis=0, keepdims=True)
        s2 = jnp.sum(X * X, axis=0, keepdims=True)
        new = jnp.concatenate([s1, s2], axis=0)

        @pl.when(i == 0)
        def _():
            out_ref[0] = new

        @pl.when(i > 0)
        def _():
            out_ref[0] = out_ref[0] + new


def _conv_layer(inputs, p, cout):
    B = inputs[0][0].shape[0]
    R = _CONV_R
    grid = (B, NK // R)
    t_flags = tuple(t for _, _, t in inputs)

    def wmap(stage, with_g3):
        arrs = [p["b1"].reshape(1, cout)]
        for j in (2, 3):
            if stage >= j:
                arrs += [p["g%d" % (j - 1)].reshape(1, cout),
                         p["be%d" % (j - 1)].reshape(1, cout),
                         p["W%d" % j], p["b%d" % j].reshape(1, cout)]
        if with_g3:
            arrs += [p["g3"].reshape(1, cout), p["be3"].reshape(1, cout)]
        return arrs

    def full_spec(a):
        nd = a.ndim
        return pl.BlockSpec(a.shape, lambda b, i, _nd=nd: (0,) * _nd)

    in_arrs = []
    in_specs = []
    for a, w, is_t in inputs:
        in_arrs += [a, w]
        if is_t:
            spec = pl.BlockSpec((1, a.shape[1], R), lambda b, i: (b, 0, i))
        else:
            spec = pl.BlockSpec((1, R, a.shape[2]), lambda b, i: (b, i, 0))
        in_specs += [spec, full_spec(w)]
    stat_spec = pl.BlockSpec((1, 2, cout), lambda b, i: (b, 0, 0))

    stats = []
    for stage in (1, 2, 3, 4):
        pool = stage == 4
        s = min(stage, 3)
        warrs = wmap(s, pool)
        body = functools.partial(_pass_body, s, t_flags, pool)

        def wrapped(*refs, _body=body):
            _body(refs)

        n_stats = s - 1 + (1 if pool else 0)
        if pool:
            out_spec = pl.BlockSpec((1, R // K, cout), lambda b, i: (b, i, 0))
            out_shape = jax.ShapeDtypeStruct((B, N, cout), jnp.float32)
        else:
            out_spec = stat_spec
            out_shape = jax.ShapeDtypeStruct((B, 2, cout), jnp.float32)
        res = pl.pallas_call(
            wrapped,
            grid=grid,
            in_specs=(in_specs + [stat_spec] * n_stats
                      + [full_spec(a) for a in warrs]),
            out_specs=out_spec,
            out_shape=out_shape,
        )(*in_arrs, *stats[:n_stats], *warrs)
        if pool:
            return res
        stats.append(res)


def kernel(pc, params):
    B = pc.shape[0]
    pcT = jnp.swapaxes(pc, 1, 2)
    nbr = _knn(pc, pcT)
    edges = nbr.reshape(-1)
    pc_pad = jnp.pad(pc.reshape(B * N, 3), ((0, 0), (0, 125)))
    xyzg = _sc_gather(pc_pad, edges).reshape(B, NK, 128)
    GE8, E3 = _edge_prep(xyzg, pc)
    edge_feats = E3.reshape(B * NK, 3)
    GET = jnp.swapaxes(GE8, 1, 2)

    def we8(W, c_in):
        z = jnp.zeros((8, W.shape[1]), jnp.float32)
        return z.at[3:6].set(W[c_in:])

    W1 = params["conv1"]["W1"]
    Wc1 = jnp.pad(W1, ((0, 2), (0, 0)))
    pooled1 = _conv_layer([(GET, Wc1, True)], params["conv1"], 32)

    t1 = jnp.pad(pooled1.reshape(B * N, 32), ((0, 0), (0, 96)))
    G2 = _sc_gather(t1, edges).reshape(B, NK, 128)
    W2 = params["conv2"]["W1"]
    pooled2 = _conv_layer(
        [(G2, jnp.pad(W2[:32], ((0, 96), (0, 0))), False),
         (GET, we8(W2, 32), True)],
        params["conv2"], 64)

    t2 = jnp.pad(pooled2.reshape(B * N, 64), ((0, 0), (0, 64)))
    G3 = _sc_gather(t2, edges).reshape(B, NK, 128)
    W3 = params["conv3"]["W1"]
    pooled3 = _conv_layer(
        [(G3, jnp.pad(W3[:64], ((0, 64), (0, 0))), False),
         (GET, we8(W3, 64), True)],
        params["conv3"], 128)

    x = jnp.swapaxes(pooled3, 1, 2)
    return (x, edges, edge_feats)


# speedup vs baseline: 4.2494x; 1.0913x over previous
import functools

import jax
import jax.numpy as jnp
from jax import lax
from jax.experimental import pallas as pl
from jax.experimental.pallas import tpu as pltpu
from jax.experimental.pallas import tpu_sc as plsc

K = 32
N = 4096
NK = N * K


_KNN_R = 256


def _knn_body(rows_ref, pcT_ref, nbr_ref, d_ref):
    b = pl.program_id(0)
    rows = rows_ref[0]
    pcT = pcT_ref[0]
    R = rows.shape[0]

    r0 = rows[:, 0:1]
    r1 = rows[:, 1:2]
    r2 = rows[:, 2:3]
    sq_r = (r0 * r0 + r1 * r1) + r2 * r2
    c0 = pcT[0:1, :]
    c1 = pcT[1:2, :]
    c2 = pcT[2:3, :]
    sq_c = (c0 * c0 + c1 * c1) + c2 * c2

    mm = lax.dot_general(rows, pcT, (((1,), (0,)), ((), ())),
                         preferred_element_type=jnp.float32)
    d_ref[...] = (sq_r + sq_c) - 2.0 * mm

    NV = N // 128
    lane_f = lax.broadcasted_iota(jnp.int32, (R, 128), 1).astype(jnp.float32)
    iota_k = lax.broadcasted_iota(jnp.int32, (R, K), 1)
    inf = jnp.float32(jnp.inf)
    bigN = jnp.float32(N)

    def it(t, carry):
        jf_prev, nbr_acc = carry
        v_acc = jnp.full((R, 128), inf, jnp.float32)
        k_acc = jnp.zeros((R, 128), jnp.float32)
        for k in range(NV):
            dk = d_ref[:, k * 128:(k + 1) * 128]
            hit = lane_f == (jf_prev - jnp.float32(k * 128))
            dk = jnp.where(hit, inf, dk)
            d_ref[:, k * 128:(k + 1) * 128] = dk
            better = dk < v_acc
            v_acc = jnp.where(better, dk, v_acc)
            k_acc = jnp.where(better, jnp.float32(k), k_acc)
        m = jnp.min(v_acc, axis=1, keepdims=True)
        jf = jnp.min(jnp.where(v_acc == m, k_acc * 128.0 + lane_f, bigN),
                     axis=1, keepdims=True)
        nbr_acc = jnp.where(iota_k == t, jf.astype(jnp.int32), nbr_acc)
        return jf, nbr_acc

    jf0 = jnp.full((R, 1), -1.0, jnp.float32)
    nbr0 = jnp.zeros((R, K), jnp.int32)
    _, nbr_acc = lax.fori_loop(0, K, it, (jf0, nbr0))
    nbr_ref[0] = nbr_acc + b * N


def _knn(pc, pcT):
    B = pc.shape[0]
    R = _KNN_R
    grid = (B, N // R)
    return pl.pallas_call(
        _knn_body,
        grid=grid,
        in_specs=[
            pl.BlockSpec((1, R, 3), lambda b, i: (b, i, 0)),
            pl.BlockSpec((1, 3, N), lambda b, i: (b, 0, 0)),
        ],
        out_specs=pl.BlockSpec((1, R, K), lambda b, i: (b, i, 0)),
        out_shape=jax.ShapeDtypeStruct((B, N, K), jnp.int32),
        scratch_shapes=[pltpu.VMEM((R, N), jnp.float32)],
    )(pc, pcT)


_EP_R = 2048


def _edge_prep_body(xg_ref, ctr_ref, ge_ref, ef_ref):
    xg = xg_ref[0][:, :3]
    ctr = ctr_ref[0]
    rep = jnp.broadcast_to(ctr[:, None, :], (ctr.shape[0], K, 3))
    rep = rep.reshape(xg.shape[0], 3)
    ef = xg - rep
    zero2 = jnp.zeros((xg.shape[0], 2), jnp.float32)
    ge_ref[0] = jnp.concatenate([xg, ef, zero2], axis=1)
    ef_ref[0] = ef


def _edge_prep(xyzg, pc):
    B = pc.shape[0]
    R = _EP_R
    grid = (B, NK // R)
    return pl.pallas_call(
        _edge_prep_body,
        grid=grid,
        in_specs=[
            pl.BlockSpec((1, R, 128), lambda b, i: (b, i, 0)),
            pl.BlockSpec((1, R // K, 3), lambda b, i: (b, i, 0)),
        ],
        out_specs=[
            pl.BlockSpec((1, R, 8), lambda b, i: (b, i, 0)),
            pl.BlockSpec((1, R, 3), lambda b, i: (b, i, 0)),
        ],
        out_shape=[
            jax.ShapeDtypeStruct((B, NK, 8), jnp.float32),
            jax.ShapeDtypeStruct((B, NK, 3), jnp.float32),
        ],
    )(xyzg, pc)


_SC_CHUNK = 128
_SC_NBUF = 4


def _sc_gather(table, idx):
    M = idx.shape[0]
    D = table.shape[1]
    info = plsc.get_sparse_core_info()
    NW = info.num_cores * info.num_subcores
    m_per_w = M // NW
    CH = _SC_CHUNK
    NB = _SC_NBUF
    n_iter = m_per_w // CH
    mesh = plsc.VectorSubcoreMesh(core_axis_name="c", subcore_axis_name="s")

    @functools.partial(
        pl.kernel,
        out_type=jax.ShapeDtypeStruct((M, D), jnp.float32),
        mesh=mesh,
        scratch_types=[
            pltpu.VMEM((NB, CH), jnp.int32),
            pltpu.VMEM((NB, CH, D), jnp.float32),
            [pltpu.SemaphoreType.DMA] * NB,
        ],
    )
    def gather_kernel(table_hbm, idx_hbm, out_hbm, idx_v, rows_v, sems):
        wid = lax.axis_index("s") * info.num_cores + lax.axis_index("c")
        base = wid * m_per_w
        copies = [None] * NB

        def start(i, buf):
            off = base + i * CH
            pltpu.sync_copy(idx_hbm.at[pl.ds(off, CH)], idx_v.at[buf])
            copies[buf] = pltpu.async_copy(
                table_hbm.at[idx_v.at[buf]], rows_v.at[buf], sems[buf])

        for i in range(min(NB, n_iter)):
            start(i, i)
        for i in range(n_iter):
            buf = i % NB
            copies[buf].wait()
            pltpu.sync_copy(rows_v.at[buf], out_hbm.at[pl.ds(base + i * CH, CH)])
            if i + NB < n_iter:
                start(i + NB, buf)

    return gather_kernel(table, idx)


_CONV_R = 4096
_EPS = 1e-5


def _lrelu(x):
    return jnp.where(x >= 0, x, 0.1 * x)


def _norm(x, stats_blk, g, be):
    m_rows = jnp.float32(NK)
    s1 = stats_blk[0:1, :]
    s2 = stats_blk[1:2, :]
    mean = s1 / m_rows
    var = s2 / m_rows - mean * mean
    inv = lax.rsqrt(var + _EPS)
    return (x - mean) * inv * g + be


def _dot(a, w, is_t):
    if is_t:
        return lax.dot_general(a, w, (((0,), (0,)), ((), ())),
                               preferred_element_type=jnp.float32)
    return lax.dot_general(a, w, (((1,), (0,)), ((), ())),
                           preferred_element_type=jnp.float32)


def _chain(ins, ws, stats, upto):
    X = sum(_dot(a, w, t) for a, w, t in ins) + ws["b1"]
    for j in (2, 3):
        if upto < j:
            break
        Xn = _lrelu(_norm(X, stats[j - 2], ws["g%d" % (j - 1)], ws["be%d" % (j - 1)]))
        X = _dot(Xn, ws["W%d" % j], False) + ws["b%d" % j]
    return X


def _w_names(stage):
    names = ["b1"]
    for j in (2, 3):
        if stage >= j:
            names += ["g%d" % (j - 1), "be%d" % (j - 1), "W%d" % j, "b%d" % j]
    return names


def _pass_body(stage, t_flags, pool, refs):
    i = pl.program_id(1)
    n_in = len(t_flags)
    ins = [(refs[2 * k][0], refs[2 * k + 1][...], t_flags[k])
           for k in range(n_in)]
    pos = 2 * n_in
    n_stats = stage - 1 + (1 if pool else 0)
    stats = [refs[pos + k][0] for k in range(n_stats)]
    pos += n_stats
    w_refs = refs[pos:-1]
    out_ref = refs[-1]
    names = _w_names(stage) + (["g3", "be3"] if pool else [])
    ws = {n: w_refs[k][...] for k, n in enumerate(names)}
    X = _chain(ins, ws, stats, stage)
    if pool:
        Xn = _lrelu(_norm(X, stats[-1], ws["g3"], ws["be3"]))
        c = Xn.shape[-1]
        P = jnp.max(Xn.reshape(Xn.shape[0] // K, K, c), axis=1)
        out_ref[0] = P
    else:
        s1 = jnp.sum(X, axis=0, keepdims=True)
        s2 = jnp.sum(X * X, axis=0, keepdims=True)
        new = jnp.concatenate([s1, s2], axis=0)

        @pl.when(i == 0)
        def _():
            out_ref[0] = new

        @pl.when(i > 0)
        def _():
            out_ref[0] = out_ref[0] + new


def _conv_layer(inputs, p, cout):
    B = inputs[0][0].shape[0]
    R = _CONV_R
    grid = (B, NK // R)
    t_flags = tuple(t for _, _, t in inputs)

    def wmap(stage, with_g3):
        arrs = [p["b1"].reshape(1, cout)]
        for j in (2, 3):
            if stage >= j:
                arrs += [p["g%d" % (j - 1)].reshape(1, cout),
                         p["be%d" % (j - 1)].reshape(1, cout),
                         p["W%d" % j], p["b%d" % j].reshape(1, cout)]
        if with_g3:
            arrs += [p["g3"].reshape(1, cout), p["be3"].reshape(1, cout)]
        return arrs

    def full_spec(a):
        nd = a.ndim
        return pl.BlockSpec(a.shape, lambda b, i, _nd=nd: (0,) * _nd)

    in_arrs = []
    in_specs = []
    for a, w, is_t in inputs:
        in_arrs += [a, w]
        if is_t:
            spec = pl.BlockSpec((1, a.shape[1], R), lambda b, i: (b, 0, i))
        else:
            spec = pl.BlockSpec((1, R, a.shape[2]), lambda b, i: (b, i, 0))
        in_specs += [spec, full_spec(w)]
    stat_spec = pl.BlockSpec((1, 2, cout), lambda b, i: (b, 0, 0))

    stats = []
    for stage in (1, 2, 3, 4):
        pool = stage == 4
        s = min(stage, 3)
        warrs = wmap(s, pool)
        body = functools.partial(_pass_body, s, t_flags, pool)

        def wrapped(*refs, _body=body):
            _body(refs)

        n_stats = s - 1 + (1 if pool else 0)
        if pool:
            out_spec = pl.BlockSpec((1, R // K, cout), lambda b, i: (b, i, 0))
            out_shape = jax.ShapeDtypeStruct((B, N, cout), jnp.float32)
        else:
            out_spec = stat_spec
            out_shape = jax.ShapeDtypeStruct((B, 2, cout), jnp.float32)
        res = pl.pallas_call(
            wrapped,
            grid=grid,
            in_specs=(in_specs + [stat_spec] * n_stats
                      + [full_spec(a) for a in warrs]),
            out_specs=out_spec,
            out_shape=out_shape,
        )(*in_arrs, *stats[:n_stats], *warrs)
        if pool:
            return res
        stats.append(res)


def kernel(pc, params):
    B = pc.shape[0]
    pcT = jnp.swapaxes(pc, 1, 2)
    nbr = _knn(pc, pcT)
    edges = nbr.reshape(-1)
    pc_pad = jnp.pad(pc.reshape(B * N, 3), ((0, 0), (0, 125)))
    xyzg = _sc_gather(pc_pad, edges).reshape(B, NK, 128)
    GE8, E3 = _edge_prep(xyzg, pc)
    edge_feats = E3.reshape(B * NK, 3)
    GET = jnp.swapaxes(GE8, 1, 2)

    def we8(W, c_in):
        z = jnp.zeros((8, W.shape[1]), jnp.float32)
        return z.at[3:6].set(W[c_in:])

    W1 = params["conv1"]["W1"]
    Wc1 = jnp.pad(W1, ((0, 2), (0, 0)))
    pooled1 = _conv_layer([(GET, Wc1, True)], params["conv1"], 32)

    t1 = jnp.pad(pooled1.reshape(B * N, 32), ((0, 0), (0, 96)))
    G2 = _sc_gather(t1, edges).reshape(B, NK, 128)
    W2 = params["conv2"]["W1"]
    pooled2 = _conv_layer(
        [(G2, jnp.pad(W2[:32], ((0, 96), (0, 0))), False),
         (GET, we8(W2, 32), True)],
        params["conv2"], 64)

    t2 = jnp.pad(pooled2.reshape(B * N, 64), ((0, 0), (0, 64)))
    G3 = _sc_gather(t2, edges).reshape(B, NK, 128)
    W3 = params["conv3"]["W1"]
    pooled3 = _conv_layer(
        [(G3, jnp.pad(W3[:64], ((0, 64), (0, 0))), False),
         (GET, we8(W3, 64), True)],
        params["conv3"], 128)

    x = jnp.swapaxes(pooled3, 1, 2)
    return (x, edges, edge_feats)


# speedup vs baseline: 4.4044x; 1.0365x over previous
import functools

import jax
import jax.numpy as jnp
from jax import lax
from jax.experimental import pallas as pl
from jax.experimental.pallas import tpu as pltpu
from jax.experimental.pallas import tpu_sc as plsc

K = 32
N = 4096
NK = N * K


_KNN_R = 256


def _knn_body(rows_ref, pcT_ref, nbr_ref, d_ref):
    b = pl.program_id(0)
    rows = rows_ref[0]
    pcT = pcT_ref[0]
    R = rows.shape[0]

    r0 = rows[:, 0:1]
    r1 = rows[:, 1:2]
    r2 = rows[:, 2:3]
    sq_r = (r0 * r0 + r1 * r1) + r2 * r2
    c0 = pcT[0:1, :]
    c1 = pcT[1:2, :]
    c2 = pcT[2:3, :]
    sq_c = (c0 * c0 + c1 * c1) + c2 * c2

    mm = lax.dot_general(rows, pcT, (((1,), (0,)), ((), ())),
                         preferred_element_type=jnp.float32)
    d_ref[...] = (sq_r + sq_c) - 2.0 * mm

    NV = N // 128
    lane_f = lax.broadcasted_iota(jnp.int32, (R, 128), 1).astype(jnp.float32)
    iota_k = lax.broadcasted_iota(jnp.int32, (R, K), 1)
    inf = jnp.float32(jnp.inf)
    bigN = jnp.float32(N)

    def it(t, carry):
        jf_prev, nbr_acc = carry
        v_acc = jnp.full((R, 128), inf, jnp.float32)
        k_acc = jnp.zeros((R, 128), jnp.float32)
        for k in range(NV):
            dk = d_ref[:, k * 128:(k + 1) * 128]
            hit = lane_f == (jf_prev - jnp.float32(k * 128))
            dk = jnp.where(hit, inf, dk)
            d_ref[:, k * 128:(k + 1) * 128] = dk
            better = dk < v_acc
            v_acc = jnp.where(better, dk, v_acc)
            k_acc = jnp.where(better, jnp.float32(k), k_acc)
        m = jnp.min(v_acc, axis=1, keepdims=True)
        jf = jnp.min(jnp.where(v_acc == m, k_acc * 128.0 + lane_f, bigN),
                     axis=1, keepdims=True)
        nbr_acc = jnp.where(iota_k == t, jf.astype(jnp.int32), nbr_acc)
        return jf, nbr_acc

    jf0 = jnp.full((R, 1), -1.0, jnp.float32)
    nbr0 = jnp.zeros((R, K), jnp.int32)
    _, nbr_acc = lax.fori_loop(0, K, it, (jf0, nbr0))
    nbr_ref[0] = nbr_acc + b * N


def _knn(pc, pcT):
    B = pc.shape[0]
    R = _KNN_R
    grid = (B, N // R)
    return pl.pallas_call(
        _knn_body,
        grid=grid,
        in_specs=[
            pl.BlockSpec((1, R, 3), lambda b, i: (b, i, 0)),
            pl.BlockSpec((1, 3, N), lambda b, i: (b, 0, 0)),
        ],
        out_specs=pl.BlockSpec((1, R, K), lambda b, i: (b, i, 0)),
        out_shape=jax.ShapeDtypeStruct((B, N, K), jnp.int32),
        scratch_shapes=[pltpu.VMEM((R, N), jnp.float32)],
    )(pc, pcT)


_EP_R = 2048


def _edge_prep_body(xg_ref, ctr_ref, ge_ref, ef_ref):
    xg = xg_ref[0][:, :3]
    ctr = ctr_ref[0]
    rep = jnp.broadcast_to(ctr[:, None, :], (ctr.shape[0], K, 3))
    rep = rep.reshape(xg.shape[0], 3)
    ef = xg - rep
    zero2 = jnp.zeros((xg.shape[0], 2), jnp.float32)
    ge_ref[0] = jnp.concatenate([xg, ef, zero2], axis=1)
    ef_ref[0] = ef


def _edge_prep(xyzg, pc):
    B = pc.shape[0]
    R = _EP_R
    grid = (B, NK // R)
    return pl.pallas_call(
        _edge_prep_body,
        grid=grid,
        in_specs=[
            pl.BlockSpec((1, R, 128), lambda b, i: (b, i, 0)),
            pl.BlockSpec((1, R // K, 3), lambda b, i: (b, i, 0)),
        ],
        out_specs=[
            pl.BlockSpec((1, R, 8), lambda b, i: (b, i, 0)),
            pl.BlockSpec((1, R, 3), lambda b, i: (b, i, 0)),
        ],
        out_shape=[
            jax.ShapeDtypeStruct((B, NK, 8), jnp.float32),
            jax.ShapeDtypeStruct((B, NK, 3), jnp.float32),
        ],
    )(xyzg, pc)


_SC_CHUNK = 128
_SC_NBUF = 4


def _sc_gather(table, idx):
    M = idx.shape[0]
    D = table.shape[1]
    info = plsc.get_sparse_core_info()
    NW = info.num_cores * info.num_subcores
    m_per_w = M // NW
    CH = _SC_CHUNK
    NB = _SC_NBUF
    n_iter = m_per_w // CH
    mesh = plsc.VectorSubcoreMesh(core_axis_name="c", subcore_axis_name="s")

    @functools.partial(
        pl.kernel,
        out_type=jax.ShapeDtypeStruct((M, D), jnp.float32),
        mesh=mesh,
        scratch_types=[
            pltpu.VMEM((NB, CH), jnp.int32),
            pltpu.VMEM((NB, CH, D), jnp.float32),
            [pltpu.SemaphoreType.DMA] * NB,
        ],
    )
    def gather_kernel(table_hbm, idx_hbm, out_hbm, idx_v, rows_v, sems):
        wid = lax.axis_index("s") * info.num_cores + lax.axis_index("c")
        base = wid * m_per_w
        copies = [None] * NB

        def start(i, buf):
            off = base + i * CH
            pltpu.sync_copy(idx_hbm.at[pl.ds(off, CH)], idx_v.at[buf])
            copies[buf] = pltpu.async_copy(
                table_hbm.at[idx_v.at[buf]], rows_v.at[buf], sems[buf])

        for i in range(min(NB, n_iter)):
            start(i, i)
        for i in range(n_iter):
            buf = i % NB
            copies[buf].wait()
            pltpu.sync_copy(rows_v.at[buf], out_hbm.at[pl.ds(base + i * CH, CH)])
            if i + NB < n_iter:
                start(i + NB, buf)

    return gather_kernel(table, idx)


_CONV_R = 8192
_EPS = 1e-5


def _lrelu(x):
    return jnp.where(x >= 0, x, 0.1 * x)


def _norm(x, stats_blk, g, be):
    m_rows = jnp.float32(NK)
    s1 = stats_blk[0:1, :]
    s2 = stats_blk[1:2, :]
    mean = s1 / m_rows
    var = s2 / m_rows - mean * mean
    inv = lax.rsqrt(var + _EPS)
    return (x - mean) * inv * g + be


def _dot(a, w, is_t):
    if is_t:
        return lax.dot_general(a, w, (((0,), (0,)), ((), ())),
                               preferred_element_type=jnp.float32)
    return lax.dot_general(a, w, (((1,), (0,)), ((), ())),
                           preferred_element_type=jnp.float32)


def _chain(ins, ws, stats, upto):
    X = sum(_dot(a, w, t) for a, w, t in ins) + ws["b1"]
    for j in (2, 3):
        if upto < j:
            break
        Xn = _lrelu(_norm(X, stats[j - 2], ws["g%d" % (j - 1)], ws["be%d" % (j - 1)]))
        X = _dot(Xn, ws["W%d" % j], False) + ws["b%d" % j]
    return X


def _w_names(stage):
    names = ["b1"]
    for j in (2, 3):
        if stage >= j:
            names += ["g%d" % (j - 1), "be%d" % (j - 1), "W%d" % j, "b%d" % j]
    return names


def _pass_body(stage, t_flags, pool, refs):
    i = pl.program_id(1)
    n_in = len(t_flags)
    ins = [(refs[2 * k][0], refs[2 * k + 1][...], t_flags[k])
           for k in range(n_in)]
    pos = 2 * n_in
    n_stats = stage - 1 + (1 if pool else 0)
    stats = [refs[pos + k][0] for k in range(n_stats)]
    pos += n_stats
    w_refs = refs[pos:-1]
    out_ref = refs[-1]
    names = _w_names(stage) + (["g3", "be3"] if pool else [])
    ws = {n: w_refs[k][...] for k, n in enumerate(names)}
    X = _chain(ins, ws, stats, stage)
    if pool:
        Xn = _lrelu(_norm(X, stats[-1], ws["g3"], ws["be3"]))
        c = Xn.shape[-1]
        P = jnp.max(Xn.reshape(Xn.shape[0] // K, K, c), axis=1)
        out_ref[0] = P
    else:
        s1 = jnp.sum(X, axis=0, keepdims=True)
        s2 = jnp.sum(X * X, axis=0, keepdims=True)
        new = jnp.concatenate([s1, s2], axis=0)

        @pl.when(i == 0)
        def _():
            out_ref[0] = new

        @pl.when(i > 0)
        def _():
            out_ref[0] = out_ref[0] + new


def _conv_layer(inputs, p, cout):
    B = inputs[0][0].shape[0]
    R = _CONV_R
    grid = (B, NK // R)
    t_flags = tuple(t for _, _, t in inputs)

    def wmap(stage, with_g3):
        arrs = [p["b1"].reshape(1, cout)]
        for j in (2, 3):
            if stage >= j:
                arrs += [p["g%d" % (j - 1)].reshape(1, cout),
                         p["be%d" % (j - 1)].reshape(1, cout),
                         p["W%d" % j], p["b%d" % j].reshape(1, cout)]
        if with_g3:
            arrs += [p["g3"].reshape(1, cout), p["be3"].reshape(1, cout)]
        return arrs

    def full_spec(a):
        nd = a.ndim
        return pl.BlockSpec(a.shape, lambda b, i, _nd=nd: (0,) * _nd)

    in_arrs = []
    in_specs = []
    for a, w, is_t in inputs:
        in_arrs += [a, w]
        if is_t:
            spec = pl.BlockSpec((1, a.shape[1], R), lambda b, i: (b, 0, i))
        else:
            spec = pl.BlockSpec((1, R, a.shape[2]), lambda b, i: (b, i, 0))
        in_specs += [spec, full_spec(w)]
    stat_spec = pl.BlockSpec((1, 2, cout), lambda b, i: (b, 0, 0))

    stats = []
    for stage in (1, 2, 3, 4):
        pool = stage == 4
        s = min(stage, 3)
        warrs = wmap(s, pool)
        body = functools.partial(_pass_body, s, t_flags, pool)

        def wrapped(*refs, _body=body):
            _body(refs)

        n_stats = s - 1 + (1 if pool else 0)
        if pool:
            out_spec = pl.BlockSpec((1, R // K, cout), lambda b, i: (b, i, 0))
            out_shape = jax.ShapeDtypeStruct((B, N, cout), jnp.float32)
        else:
            out_spec = stat_spec
            out_shape = jax.ShapeDtypeStruct((B, 2, cout), jnp.float32)
        res = pl.pallas_call(
            wrapped,
            grid=grid,
            in_specs=(in_specs + [stat_spec] * n_stats
                      + [full_spec(a) for a in warrs]),
            out_specs=out_spec,
            out_shape=out_shape,
        )(*in_arrs, *stats[:n_stats], *warrs)
        if pool:
            return res
        stats.append(res)


def kernel(pc, params):
    B = pc.shape[0]
    pcT = jnp.swapaxes(pc, 1, 2)
    nbr = _knn(pc, pcT)
    edges = nbr.reshape(-1)
    pc_pad = jnp.pad(pc.reshape(B * N, 3), ((0, 0), (0, 125)))
    xyzg = _sc_gather(pc_pad, edges).reshape(B, NK, 128)
    GE8, E3 = _edge_prep(xyzg, pc)
    edge_feats = E3.reshape(B * NK, 3)
    GET = jnp.swapaxes(GE8, 1, 2)

    def we8(W, c_in):
        z = jnp.zeros((8, W.shape[1]), jnp.float32)
        return z.at[3:6].set(W[c_in:])

    W1 = params["conv1"]["W1"]
    Wc1 = jnp.pad(W1, ((0, 2), (0, 0)))
    pooled1 = _conv_layer([(GET, Wc1, True)], params["conv1"], 32)

    t1 = jnp.pad(pooled1.reshape(B * N, 32), ((0, 0), (0, 96)))
    G2 = _sc_gather(t1, edges).reshape(B, NK, 128)
    W2 = params["conv2"]["W1"]
    pooled2 = _conv_layer(
        [(G2, jnp.pad(W2[:32], ((0, 96), (0, 0))), False),
         (GET, we8(W2, 32), True)],
        params["conv2"], 64)

    t2 = jnp.pad(pooled2.reshape(B * N, 64), ((0, 0), (0, 64)))
    G3 = _sc_gather(t2, edges).reshape(B, NK, 128)
    W3 = params["conv3"]["W1"]
    pooled3 = _conv_layer(
        [(G3, jnp.pad(W3[:64], ((0, 64), (0, 0))), False),
         (GET, we8(W3, 64), True)],
        params["conv3"], 128)

    x = jnp.swapaxes(pooled3, 1, 2)
    return (x, edges, edge_feats)
